# Initial kernel scaffold; baseline (speedup 1.0000x reference)
import jax
import jax.numpy as jnp
from jax.experimental import pallas as pl


def kernel(t, W, time_embedding):
    raise NotImplementedError("write your pallas kernel here")


# speedup vs baseline: 1.8582x; 1.8582x over previous
import functools

import jax
import jax.numpy as jnp
from jax import lax
from jax.experimental import pallas as pl
from jax.experimental.pallas import tpu as pltpu
from jax.experimental.pallas import tpu_sc as plsc

_LANES = 16


def _offsets(n, csz):
    offs = list(range(0, n - csz, csz))
    offs.append(n - csz)
    return offs


@functools.lru_cache(maxsize=None)
def _build(B, N, D):
    info = plsc.get_sparse_core_info()
    nc, ns = info.num_cores, info.num_subcores
    nw = nc * ns
    bpw = B // nw
    csz = 128
    offs = _offsets(N, csz)

    mesh = plsc.VectorSubcoreMesh(core_axis_name="c", subcore_axis_name="s")

    @functools.partial(
        pl.kernel,
        out_type=jax.ShapeDtypeStruct((B, N, D), jnp.float32),
        mesh=mesh,
        compiler_params=pltpu.CompilerParams(use_tc_tiling_on_sc=False),
        scratch_types=[
            pltpu.VMEM((bpw, N), jnp.int32),
            pltpu.VMEM((csz, D), jnp.float32),
            pltpu.VMEM((csz, D), jnp.float32),
            pltpu.SemaphoreType.DMA,
        ],
    )
    def emb_kernel(t_hbm, w_hbm, te_hbm, out_hbm, t_v, te_v, rows_v, sem):
        wid = lax.axis_index("s") * nc + lax.axis_index("c")
        b0 = wid * bpw
        pltpu.sync_copy(t_hbm.at[pl.ds(b0, bpw), :], t_v)
        for off in offs:
            pltpu.sync_copy(te_hbm.at[pl.ds(off, csz), :], te_v)
            for b in range(bpw):
                idx = t_v.at[b, pl.ds(off, csz)]
                pltpu.async_copy(w_hbm.at[idx], rows_v, sem).wait()

                def body(r, _):
                    for j in range(D // _LANES):
                        s = pl.ds(j * _LANES, _LANES)
                        rows_v[r, s] = rows_v[r, s] * te_v[r, s]
                    return 0

                lax.fori_loop(0, csz, body, 0, unroll=2)
                pltpu.sync_copy(rows_v, out_hbm.at[b0 + b, pl.ds(off, csz), :])

    return emb_kernel


def kernel(t, W, time_embedding):
    B, N = t.shape
    D = W.shape[1]
    return _build(B, N, D)(t, W, time_embedding)


# speedup vs baseline: 4.8617x; 2.6164x over previous
import functools

import jax
import jax.numpy as jnp
from jax import lax
from jax.experimental import pallas as pl
from jax.experimental.pallas import tpu as pltpu
from jax.experimental.pallas import tpu_sc as plsc

_LANES = 16


@functools.lru_cache(maxsize=None)
def _build(B, N, D):
    info = plsc.get_sparse_core_info()
    nc, ns = info.num_cores, info.num_subcores
    nw = nc * ns
    bpw = B // nw
    csz = 128
    nchunks = (N + csz - 1) // csz

    mesh = plsc.VectorSubcoreMesh(core_axis_name="c", subcore_axis_name="s")

    @functools.partial(
        pl.kernel,
        out_type=jax.ShapeDtypeStruct((B, N, D), jnp.float32),
        mesh=mesh,
        compiler_params=pltpu.CompilerParams(use_tc_tiling_on_sc=False),
        scratch_types=[
            pltpu.VMEM((bpw, N), jnp.int32),
            pltpu.VMEM((csz, D), jnp.float32),
            pltpu.VMEM((csz, D), jnp.float32),
            pltpu.VMEM((csz, D), jnp.float32),
            pltpu.SemaphoreType.DMA,
            pltpu.SemaphoreType.DMA,
            pltpu.SemaphoreType.DMA,
            pltpu.SemaphoreType.DMA,
        ],
    )
    def emb_kernel(t_hbm, w_hbm, te_hbm, out_hbm, t_v, te_v, r0, r1,
                   gs0, gs1, ws0, ws1):
        rows = (r0, r1)
        gsem = (gs0, gs1)
        wsem = (ws0, ws1)
        wid = lax.axis_index("s") * nc + lax.axis_index("c")
        b0 = wid * bpw
        pltpu.sync_copy(t_hbm.at[pl.ds(b0, bpw), :], t_v)

        def chunk_off(c):
            return lax.min(c * csz, N - csz)

        def issue_gather(c, b, buf):
            idx = t_v.at[b, pl.ds(chunk_off(c), csz)]
            pltpu.async_copy(w_hbm.at[idx], rows[buf], gsem[buf])

        def wait_gather(buf):
            pltpu.make_async_copy(
                w_hbm.at[t_v.at[0, pl.ds(0, csz)]], rows[buf],
                gsem[buf]).wait()

        def wait_write(buf):
            pltpu.make_async_copy(
                rows[buf], out_hbm.at[b0, pl.ds(0, csz), :],
                wsem[buf]).wait()

        issue_gather(0, 0, 0)

        def chunk_body(c, _):
            off = chunk_off(c)
            pltpu.sync_copy(te_hbm.at[pl.ds(off, csz), :], te_v)
            for b in range(bpw):
                cur = b % 2
                nxt = (b + 1) % 2
                wait_gather(cur)
                if b + 1 < bpw:
                    if b == 0:

                        @pl.when(c > 0)
                        def _():
                            wait_write(nxt)
                    else:
                        wait_write(nxt)
                    issue_gather(c, b + 1, nxt)
                else:

                    @pl.when(c + 1 < nchunks)
                    def _():
                        wait_write(nxt)
                        issue_gather(c + 1, 0, nxt)

                rbuf = rows[cur]

                @plsc.parallel_loop(0, csz, step=1, unroll=4)
                def _(r):
                    for j in range(D // _LANES):
                        s = pl.ds(j * _LANES, _LANES)
                        rbuf[r, s] = rbuf[r, s] * te_v[r, s]

                pltpu.async_copy(
                    rbuf, out_hbm.at[b0 + b, pl.ds(off, csz), :], wsem[cur])
            return 0

        lax.fori_loop(0, nchunks, chunk_body, 0)
        wait_write(0)
        wait_write(1)

    return emb_kernel


def kernel(t, W, time_embedding):
    B, N = t.shape
    D = W.shape[1]
    return _build(B, N, D)(t, W, time_embedding)
